# Initial kernel scaffold; baseline (speedup 1.0000x reference)
import jax
import jax.numpy as jnp
from jax.experimental import pallas as pl


def kernel(x, inter_edge_index, W, b, target_feat):
    raise NotImplementedError("write your pallas kernel here")


# speedup vs baseline: 19.3782x; 19.3782x over previous
import functools

import jax
import jax.numpy as jnp
from jax import lax
from jax.experimental import pallas as pl
from jax.experimental.pallas import tpu as pltpu
from jax.experimental.pallas import tpu_sc as plsc

N_SRC = 10000
N_TGT = 10000
N_EDGE = 320000
D = 128
TF_D = 32

NW = 32
EPT = N_EDGE // NW
CH = 128
NCH = EPT // CH
TAIL = EPT - NCH * CH
ACC_N = 10240
RPT = ACC_N // 16
ZR = 160


def _sc_mesh():
    return plsc.VectorSubcoreMesh(core_axis_name="c", subcore_axis_name="s")


def _deg_count(row_idx):
    @functools.partial(
        pl.kernel,
        out_type=jax.ShapeDtypeStruct((2, ACC_N), jnp.float32),
        mesh=_sc_mesh(),
        scratch_types=[
            pltpu.VMEM((CH,), jnp.int32),
            pltpu.VMEM((CH,), jnp.float32),
            pltpu.VMEM((RPT,), jnp.float32),
            pltpu.VMEM_SHARED((ACC_N,), jnp.float32),
            pltpu.SemaphoreType.DMA,
        ],
    )
    def k(row_hbm, out_hbm, idx_v, ones_v, zst_v, deg_sh, sem):
        c = lax.axis_index("c")
        s = lax.axis_index("s")
        for j in range(CH // 16):
            ones_v[pl.ds(j * 16, 16)] = jnp.ones((16,), jnp.float32)
        for j in range(RPT // 16):
            zst_v[pl.ds(j * 16, 16)] = jnp.zeros((16,), jnp.float32)
        pltpu.sync_copy(zst_v, deg_sh.at[pl.ds(s * RPT, RPT)])
        plsc.subcore_barrier()

        base = (c * 16 + s) * EPT

        def body(i, _):
            pltpu.sync_copy(row_hbm.at[pl.ds(base + i * CH, CH)], idx_v)
            pltpu.sync_copy(ones_v, deg_sh.at[idx_v], add=True)
            return 0

        lax.fori_loop(0, NCH, body, 0)
        pltpu.sync_copy(row_hbm.at[pl.ds(base + NCH * CH, TAIL)],
                        idx_v.at[pl.ds(0, TAIL)])
        pltpu.sync_copy(ones_v.at[pl.ds(0, TAIL)],
                        deg_sh.at[idx_v.at[pl.ds(0, TAIL)]], add=True)

        plsc.subcore_barrier()

        @pl.when(s == 0)
        def _():
            pltpu.sync_copy(deg_sh, out_hbm.at[c])

    return k(row_idx)


def _scale_matmul(p0, p1, x, w):
    def body(p0_ref, p1_ref, x_ref, w_ref, o_ref):
        dis = lax.rsqrt(p0_ref[...] + p1_ref[...] + 1.0)
        o_ref[...] = jnp.dot(x_ref[...] * dis, w_ref[...],
                             preferred_element_type=jnp.float32)

    return pl.pallas_call(
        body,
        out_shape=jax.ShapeDtypeStruct((N_SRC, D), jnp.float32),
    )(p0, p1, x, w)


def _aggregate(hs, row_idx, col_idx):
    @functools.partial(
        pl.kernel,
        out_type=jax.ShapeDtypeStruct((2, ACC_N, D), jnp.float32),
        mesh=_sc_mesh(),
        scratch_types=[
            pltpu.VMEM((CH,), jnp.int32),
            pltpu.VMEM((CH,), jnp.int32),
            pltpu.VMEM((CH, D), jnp.float32),
            pltpu.VMEM((ZR, D), jnp.float32),
            pltpu.VMEM_SHARED((ACC_N, D), jnp.float32),
            pltpu.SemaphoreType.DMA,
        ],
    )
    def k(hs_hbm, row_hbm, col_hbm, out_hbm, idx_r, idx_c, rows_v, zst_v,
          acc_sh, sem):
        c = lax.axis_index("c")
        s = lax.axis_index("s")

        def zrow(i, _):
            for j in range(D // 16):
                zst_v[i, pl.ds(j * 16, 16)] = jnp.zeros((16,), jnp.float32)
            return 0

        lax.fori_loop(0, ZR, zrow, 0)
        for q in range(RPT // ZR):
            pltpu.sync_copy(zst_v, acc_sh.at[pl.ds(s * RPT + q * ZR, ZR)])
        plsc.subcore_barrier()

        base = (c * 16 + s) * EPT

        def body(i, _):
            off = base + i * CH
            pltpu.sync_copy(row_hbm.at[pl.ds(off, CH)], idx_r)
            pltpu.sync_copy(col_hbm.at[pl.ds(off, CH)], idx_c)
            pltpu.async_copy(hs_hbm.at[idx_r], rows_v, sem).wait()
            pltpu.sync_copy(rows_v, acc_sh.at[idx_c], add=True)
            return 0

        lax.fori_loop(0, NCH, body, 0)
        off = base + NCH * CH
        pltpu.sync_copy(row_hbm.at[pl.ds(off, TAIL)], idx_r.at[pl.ds(0, TAIL)])
        pltpu.sync_copy(col_hbm.at[pl.ds(off, TAIL)], idx_c.at[pl.ds(0, TAIL)])
        pltpu.async_copy(hs_hbm.at[idx_r.at[pl.ds(0, TAIL)]],
                         rows_v.at[pl.ds(0, TAIL)], sem).wait()
        pltpu.sync_copy(rows_v.at[pl.ds(0, TAIL)],
                        acc_sh.at[idx_c.at[pl.ds(0, TAIL)]], add=True)

        plsc.subcore_barrier()
        pltpu.sync_copy(acc_sh.at[pl.ds(s * RPT, RPT)],
                        out_hbm.at[c, pl.ds(s * RPT, RPT)])

    return k(hs, row_idx, col_idx)


def _finalize(acc, b, tf):
    def body(a_ref, b_ref, t_ref, o_ref):
        y = a_ref[0, :N_TGT, :] + a_ref[1, :N_TGT, :] + b_ref[...]
        o_ref[:, :D] = jnp.maximum(y, 0.0)
        o_ref[:, D:] = jnp.abs(t_ref[...])

    return pl.pallas_call(
        body,
        out_shape=jax.ShapeDtypeStruct((N_TGT, D + TF_D), jnp.float32),
    )(acc, b, tf)


def kernel(x, inter_edge_index, W, b, target_feat):
    row = inter_edge_index[0]
    col = inter_edge_index[1]
    degp = _deg_count(row)
    p0 = degp[0, :N_SRC, None]
    p1 = degp[1, :N_SRC, None]
    hs = _scale_matmul(p0, p1, x, W)
    acc = _aggregate(hs, row, col)
    return _finalize(acc, b, target_feat)


# speedup vs baseline: 37.2318x; 1.9213x over previous
import functools

import jax
import jax.numpy as jnp
from jax import lax
from jax.experimental import pallas as pl
from jax.experimental.pallas import tpu as pltpu
from jax.experimental.pallas import tpu_sc as plsc

N_SRC = 10000
N_TGT = 10000
N_EDGE = 320000
D = 128
TF_D = 32

NW = 32
CH = 128
E_PAD = 327680
N_PADE = E_PAD - N_EDGE
NCH = E_PAD // NW // CH
SRC_N = 10016
DEG_N = 10240
DEG_RPT = DEG_N // 16
ACC_N = 10112
RPT = ACC_N // 16
PH = 2
PCH = NCH // PH


def _sc_mesh():
    return plsc.VectorSubcoreMesh(core_axis_name="c", subcore_axis_name="s")


def _deg_count(row2d):
    @functools.partial(
        pl.kernel,
        out_type=jax.ShapeDtypeStruct((2, DEG_N), jnp.float32),
        mesh=_sc_mesh(),
        scratch_types=[
            pltpu.VMEM((NCH, CH), jnp.int32),
            pltpu.VMEM((CH,), jnp.float32),
            pltpu.VMEM((DEG_RPT,), jnp.float32),
            pltpu.VMEM_SHARED((DEG_N,), jnp.float32),
            pltpu.SemaphoreType.DMA,
            pltpu.SemaphoreType.DMA,
        ],
    )
    def k(row_hbm, out_hbm, idx_v, ones_v, zst_v, deg_sh, semi, sem):
        c = lax.axis_index("c")
        s = lax.axis_index("s")
        w = c * 16 + s
        for j in range(CH // 16):
            ones_v[pl.ds(j * 16, 16)] = jnp.ones((16,), jnp.float32)
        for j in range(DEG_RPT // 16):
            zst_v[pl.ds(j * 16, 16)] = jnp.zeros((16,), jnp.float32)
        idx_cp = pltpu.async_copy(row_hbm.at[pl.ds(w * NCH, NCH)], idx_v, semi)
        pltpu.sync_copy(zst_v, deg_sh.at[pl.ds(s * DEG_RPT, DEG_RPT)])
        idx_cp.wait()
        plsc.subcore_barrier()

        def body(i, _):
            cps = [
                pltpu.async_copy(ones_v, deg_sh.at[idx_v.at[i * 8 + g]], sem,
                                 add=True)
                for g in range(8)
            ]
            for cp in cps:
                cp.wait()
            return 0

        lax.fori_loop(0, NCH // 8, body, 0)
        plsc.subcore_barrier()

        @pl.when(s == 0)
        def _():
            pltpu.sync_copy(deg_sh, out_hbm.at[c])

    return k(row2d)


def _scale_matmul(p0, p1, x, w):
    def body(p0_ref, p1_ref, x_ref, w_ref, o_ref):
        dis = lax.rsqrt(p0_ref[...] + p1_ref[...] + 1.0)
        o_ref[...] = jnp.dot(x_ref[...] * dis, w_ref[...],
                             preferred_element_type=jnp.float32)

    return pl.pallas_call(
        body,
        out_shape=jax.ShapeDtypeStruct((SRC_N, D), jnp.float32),
    )(p0, p1, x, w)


def _aggregate(hs, row2d, col2d):
    @functools.partial(
        pl.kernel,
        out_type=jax.ShapeDtypeStruct((2, ACC_N, D), jnp.float32),
        mesh=_sc_mesh(),
        scratch_types=[
            pltpu.VMEM((PCH, CH), jnp.int32),
            pltpu.VMEM((PCH, CH), jnp.int32),
            pltpu.VMEM((CH, D), jnp.float32),
            pltpu.VMEM((CH, D), jnp.float32),
            pltpu.VMEM_SHARED((ACC_N, D), jnp.float32),
            pltpu.SemaphoreType.DMA,
            pltpu.SemaphoreType.DMA,
            pltpu.SemaphoreType.DMA,
        ],
    )
    def k(hs_hbm, row_hbm, col_hbm, out_hbm, idx_r, idx_c, rows_a, rows_b,
          acc_sh, semi, sem_a, sem_b):
        c = lax.axis_index("c")
        s = lax.axis_index("s")
        w = c * 16 + s

        def zrow(i, _):
            for j in range(D // 16):
                rows_a[i, pl.ds(j * 16, 16)] = jnp.zeros((16,), jnp.float32)
            return 0

        lax.fori_loop(0, CH, zrow, 0)
        for q in range(RPT // CH):
            pltpu.sync_copy(rows_a, acc_sh.at[pl.ds(s * RPT + q * CH, CH)])
        rem = RPT - (RPT // CH) * CH
        pltpu.sync_copy(rows_a.at[pl.ds(0, rem)],
                        acc_sh.at[pl.ds(s * RPT + RPT - rem, rem)])
        plsc.subcore_barrier()

        for p in range(PH):
            pbase = w * NCH + p * PCH
            pltpu.sync_copy(row_hbm.at[pl.ds(pbase, PCH)], idx_r)
            pltpu.sync_copy(col_hbm.at[pl.ds(pbase, PCH)], idx_c)
            pltpu.async_copy(hs_hbm.at[idx_r.at[0]], rows_a, sem_a)

            def body(j, _):
                pltpu.async_copy(hs_hbm.at[idx_r.at[2 * j + 1]], rows_b, sem_b)
                pltpu.make_async_copy(hs_hbm.at[idx_r.at[2 * j]], rows_a,
                                      sem_a).wait()
                pltpu.sync_copy(rows_a, acc_sh.at[idx_c.at[2 * j]], add=True)

                @pl.when(j < PCH // 2 - 1)
                def _():
                    pltpu.async_copy(hs_hbm.at[idx_r.at[2 * j + 2]], rows_a,
                                     sem_a)

                pltpu.make_async_copy(hs_hbm.at[idx_r.at[2 * j + 1]], rows_b,
                                      sem_b).wait()
                pltpu.sync_copy(rows_b, acc_sh.at[idx_c.at[2 * j + 1]],
                                add=True)
                return 0

            lax.fori_loop(0, PCH // 2, body, 0)

        plsc.subcore_barrier()
        pltpu.sync_copy(acc_sh.at[pl.ds(s * RPT, RPT)],
                        out_hbm.at[c, pl.ds(s * RPT, RPT)])

    return k(hs, row2d, col2d)


def _finalize(acc, b, tf):
    def body(a_ref, b_ref, t_ref, o_ref):
        y = a_ref[0, :N_TGT, :] + a_ref[1, :N_TGT, :] + b_ref[...]
        o_ref[:, :D] = jnp.maximum(y, 0.0)
        o_ref[:, D:] = jnp.abs(t_ref[...])

    return pl.pallas_call(
        body,
        out_shape=jax.ShapeDtypeStruct((N_TGT, D + TF_D), jnp.float32),
    )(acc, b, tf)


def kernel(x, inter_edge_index, W, b, target_feat):
    row = inter_edge_index[0]
    col = inter_edge_index[1]
    pad_i = jnp.arange(N_PADE, dtype=row.dtype)
    row2d = jnp.concatenate([row, N_SRC + pad_i % (SRC_N - N_SRC)]).reshape(-1, CH)
    col2d = jnp.concatenate([col, N_TGT + pad_i % (ACC_N - N_TGT)]).reshape(-1, CH)
    x_pad = jnp.concatenate(
        [x, jnp.zeros((SRC_N - N_SRC, D), dtype=x.dtype)], axis=0)

    degp = _deg_count(row2d)
    p0 = degp[0, :SRC_N, None]
    p1 = degp[1, :SRC_N, None]
    hs = _scale_matmul(p0, p1, x_pad, W)
    acc = _aggregate(hs, row2d, col2d)
    return _finalize(acc, b, target_feat)


# speedup vs baseline: 38.0910x; 1.0231x over previous
import functools

import jax
import jax.numpy as jnp
from jax import lax
from jax.experimental import pallas as pl
from jax.experimental.pallas import tpu as pltpu
from jax.experimental.pallas import tpu_sc as plsc

N_SRC = 10000
N_TGT = 10000
N_EDGE = 320000
D = 128
TF_D = 32

NW = 32
CH = 128
NCHT = N_EDGE // CH
NCH = 80
NCHR = NCHT - 31 * NCH
PADR = 32 * NCH - NCHT
DEG_N = 10240
DEG_RPT = DEG_N // 16
ACC_N = 10112
RPT = ACC_N // 16
PH = 2
PCH = NCH // PH


def _sc_mesh():
    return plsc.VectorSubcoreMesh(core_axis_name="c", subcore_axis_name="s")


def _deg_count(row2d):
    @functools.partial(
        pl.kernel,
        out_type=jax.ShapeDtypeStruct((2, DEG_N), jnp.float32),
        mesh=_sc_mesh(),
        scratch_types=[
            pltpu.VMEM((NCH, CH), jnp.int32),
            pltpu.VMEM((CH,), jnp.float32),
            pltpu.VMEM((DEG_RPT,), jnp.float32),
            pltpu.VMEM_SHARED((DEG_N,), jnp.float32),
            pltpu.SemaphoreType.DMA,
            pltpu.SemaphoreType.DMA,
        ],
    )
    def k(row_hbm, out_hbm, idx_v, ones_v, zst_v, deg_sh, semi, sem):
        c = lax.axis_index("c")
        s = lax.axis_index("s")
        w = c * 16 + s
        nch_w = jnp.where(w < NW - 1, NCH, NCHR)
        for j in range(CH // 16):
            ones_v[pl.ds(j * 16, 16)] = jnp.ones((16,), jnp.float32)
        for j in range(DEG_RPT // 16):
            zst_v[pl.ds(j * 16, 16)] = jnp.zeros((16,), jnp.float32)
        idx_cp = pltpu.async_copy(row_hbm.at[pl.ds(w * NCH, NCH)], idx_v, semi)
        pltpu.sync_copy(zst_v, deg_sh.at[pl.ds(s * DEG_RPT, DEG_RPT)])
        idx_cp.wait()
        plsc.subcore_barrier()

        def body(i, _):
            cps = [
                pltpu.async_copy(ones_v, deg_sh.at[idx_v.at[i * 4 + g]], sem,
                                 add=True)
                for g in range(4)
            ]
            for cp in cps:
                cp.wait()
            return 0

        lax.fori_loop(0, nch_w // 4, body, 0)
        plsc.subcore_barrier()

        @pl.when(s == 0)
        def _():
            pltpu.sync_copy(deg_sh, out_hbm.at[c])

    return k(row2d)


def _scale_matmul(degp, x, w):
    def body(d_ref, x_ref, w_ref, o_ref):
        dis = lax.rsqrt(d_ref[0] + d_ref[1] + 1.0)
        o_ref[...] = jnp.dot(x_ref[...] * dis, w_ref[...],
                             preferred_element_type=jnp.float32)

    return pl.pallas_call(
        body,
        out_shape=jax.ShapeDtypeStruct((N_SRC, D), jnp.float32),
    )(degp, x, w)


def _aggregate(hs, row2d, col2d):
    @functools.partial(
        pl.kernel,
        out_type=jax.ShapeDtypeStruct((2, ACC_N, D), jnp.float32),
        mesh=_sc_mesh(),
        scratch_types=[
            pltpu.VMEM((PCH, CH), jnp.int32),
            pltpu.VMEM((PCH, CH), jnp.int32),
            pltpu.VMEM((CH, D), jnp.float32),
            pltpu.VMEM((CH, D), jnp.float32),
            pltpu.VMEM_SHARED((ACC_N, D), jnp.float32),
            pltpu.SemaphoreType.DMA,
            pltpu.SemaphoreType.DMA,
            pltpu.SemaphoreType.DMA,
        ],
    )
    def k(hs_hbm, row_hbm, col_hbm, out_hbm, idx_r, idx_c, rows_a, rows_b,
          acc_sh, semi, sem_a, sem_b):
        c = lax.axis_index("c")
        s = lax.axis_index("s")
        w = c * 16 + s
        nch_w = jnp.where(w < NW - 1, NCH, NCHR)

        def zrow(i, _):
            for j in range(D // 16):
                rows_a[i, pl.ds(j * 16, 16)] = jnp.zeros((16,), jnp.float32)
            return 0

        lax.fori_loop(0, CH, zrow, 0)
        for q in range(RPT // CH):
            pltpu.sync_copy(rows_a, acc_sh.at[pl.ds(s * RPT + q * CH, CH)])
        rem = RPT - (RPT // CH) * CH
        pltpu.sync_copy(rows_a.at[pl.ds(0, rem)],
                        acc_sh.at[pl.ds(s * RPT + RPT - rem, rem)])
        plsc.subcore_barrier()

        for p in range(PH):
            pairs = PCH // 2
            pltpu.sync_copy(row_hbm.at[pl.ds(w * NCH + p * PCH, PCH)], idx_r)
            pltpu.sync_copy(col_hbm.at[pl.ds(w * NCH + p * PCH, PCH)], idx_c)
            pltpu.async_copy(hs_hbm.at[idx_r.at[0]], rows_a, sem_a)

            def body(j, _):
                pltpu.async_copy(hs_hbm.at[idx_r.at[2 * j + 1]], rows_b, sem_b)
                pltpu.make_async_copy(hs_hbm.at[idx_r.at[2 * j]], rows_a,
                                      sem_a).wait()
                pltpu.sync_copy(rows_a, acc_sh.at[idx_c.at[2 * j]], add=True)

                @pl.when(j < pairs - 1)
                def _():
                    pltpu.async_copy(hs_hbm.at[idx_r.at[2 * j + 2]], rows_a,
                                     sem_a)

                pltpu.make_async_copy(hs_hbm.at[idx_r.at[2 * j + 1]], rows_b,
                                      sem_b).wait()
                pltpu.sync_copy(rows_b, acc_sh.at[idx_c.at[2 * j + 1]],
                                add=True)
                return 0

            lax.fori_loop(0, pairs, body, 0)

        plsc.subcore_barrier()
        pltpu.sync_copy(acc_sh.at[pl.ds(s * RPT, RPT)],
                        out_hbm.at[c, pl.ds(s * RPT, RPT)])

    return k(hs, row2d, col2d)


def _finalize(acc, b, tf):
    def body(a_ref, b_ref, t_ref, o_ref):
        y = a_ref[0, :N_TGT, :] + a_ref[1, :N_TGT, :] + b_ref[...]
        o_ref[:, :D] = jnp.maximum(y, 0.0)
        o_ref[:, D:] = jnp.abs(t_ref[...])

    return pl.pallas_call(
        body,
        out_shape=jax.ShapeDtypeStruct((N_TGT, D + TF_D), jnp.float32),
    )(acc, b, tf)


def kernel(x, inter_edge_index, W, b, target_feat):
    pad = jnp.arange(PADR * CH, dtype=inter_edge_index.dtype)
    row2d = jnp.concatenate([inter_edge_index[0], pad % N_SRC]).reshape(-1, CH)
    col2d = jnp.concatenate(
        [inter_edge_index[1], N_TGT + pad % (ACC_N - N_TGT)]).reshape(-1, CH)
    degp = _deg_count(row2d)
    hs = _scale_matmul(degp[:, :N_SRC, None], x, W)
    acc = _aggregate(hs, row2d, col2d)
    return _finalize(acc, b, target_feat)
